# baseline (speedup 1.0000x reference)
import jax
import jax.numpy as jnp
from jax.experimental import pallas as pl
from jax.experimental.pallas import tpu as pltpu

_RATE = 0.8
_ALPHA = 0.8
_BETA = 1.2


def _body(x_ref, o_ref, acc_ref, *, k, inv_n):
    j = pl.program_id(1)
    nj = pl.num_programs(1)

    @pl.when(j == 0)
    def _():
        acc_ref[...] = jnp.zeros_like(acc_ref)

    acc_ref[...] += jnp.sum(x_ref[...], axis=(2, 3))

    @pl.when(j == nj - 1)
    def _():
        fv = acc_ref[...] * inv_n
        gt = (fv[:, None, :] > fv[:, :, None]).astype(jnp.float32)
        cnt = jnp.sum(gt, axis=2)
        o_ref[...] = jnp.where(cnt < k, _ALPHA, _BETA)[None].astype(
            jnp.float32
        )


def kernel(inputs):
    b, c, h, w = inputs.shape
    n = h * w
    k = int(c * _RATE)

    hb = 112
    steps = h // hb

    import functools

    body = functools.partial(_body, k=k, inv_n=1.0 / n)
    out = pl.pallas_call(
        body,
        grid=(b, steps),
        in_specs=[pl.BlockSpec((1, c, hb, w), lambda i, j: (i, 0, j, 0))],
        out_specs=pl.BlockSpec((1, 1, c), lambda i, j: (i, 0, 0)),
        out_shape=jax.ShapeDtypeStruct((b, 1, c), jnp.float32),
        scratch_shapes=[pltpu.VMEM((1, c), jnp.float32)],
        compiler_params=pltpu.CompilerParams(
            dimension_semantics=("parallel", "arbitrary"),
        ),
    )(inputs)
    return out.reshape(b, c)
